# baseline (speedup 1.0000x reference)
import jax
import jax.numpy as jnp
from jax import lax
from jax.experimental import pallas as pl
from jax.experimental.pallas import tpu as pltpu
from jax.experimental.pallas import tpu_sc as plsc

_N = 10000
_NP = 10240
_E = 160000
_EP = _E // 2
_H = 64
_HP = 128
_NC = 2
_NS = 16
_NW = _NC * _NS
_CH = 128
_CHP = _CH // 2
_NCHUNKS = _E // _CH
_BN = 2048
_BEP = 4000

_f32 = jnp.float32


def _sc_mesh():
    return plsc.VectorSubcoreMesh(
        core_axis_name="c", subcore_axis_name="s",
        num_cores=_NC, num_subcores=_NS)


_GM = _NCHUNKS // _NW
_GEXTRA = _NCHUNKS - _GM * _NW
_GIDX = _GM * _CH


def _worker_id():
    return lax.axis_index("s") * _NC + lax.axis_index("c")


def _gather_body(t_hbm, dst_hbm, src_hbm, g_hbm,
                 idx_d, idx_s, rows_d0, rows_d1, rows_s0, rows_s1,
                 g_buf0, g_buf1, sem_i, sem_g0, sem_g1):
    w = _worker_id()
    start_e = w * _GIDX
    cd = pltpu.async_copy(dst_hbm.at[pl.ds(start_e, _GIDX)], idx_d, sem_i)
    cs = pltpu.async_copy(src_hbm.at[pl.ds(start_e, _GIDX)], idx_s, sem_i)
    cd.wait()
    cs.wait()
    sems = (sem_g0, sem_g1)
    rows_d = (rows_d0, rows_d1)
    rows_s = (rows_s0, rows_s1)
    g_bufs = (g_buf0, g_buf1)

    def add_chunk(b, c):
        rd, rs, gb = rows_d[b], rows_s[b], g_bufs[b]

        def add_row(j, carry2):
            for cc in range(_H // 16):
                lo = pl.ds(cc * 16, 16)
                hi = pl.ds(_H + cc * 16, 16)
                gb[j, lo] = rd[2 * j, lo] + rs[2 * j, hi]
                gb[j, hi] = rd[2 * j + 1, lo] + rs[2 * j + 1, hi]
            return carry2

        lax.fori_loop(0, _CHP, add_row, 0, unroll=4)
        base_p = (w * _GM + c) * _CHP
        pltpu.sync_copy(gb, g_hbm.at[pl.ds(base_p, _CHP)])

    def pair(t2, carry):
        for b in range(2):
            c = t2 * 2 + b

            @pl.when(c < _GM)
            def _():
                isl = pl.ds(c * _CH, _CH)
                pltpu.async_copy(t_hbm.at[idx_d.at[isl]], rows_d[b], sems[b])
                pltpu.async_copy(t_hbm.at[idx_s.at[isl]], rows_s[b], sems[b])
        for b in range(2):
            c = t2 * 2 + b

            @pl.when(c < _GM)
            def _():
                pltpu.make_async_copy(t_hbm.at[idx_d.at[pl.ds(0, _CH)]],
                                      rows_d[b], sems[b]).wait()
                pltpu.make_async_copy(t_hbm.at[idx_s.at[pl.ds(0, _CH)]],
                                      rows_s[b], sems[b]).wait()
                add_chunk(b, c)
        return carry

    lax.fori_loop(0, (_GM + 1) // 2, pair, 0)

    @pl.when(w < _GEXTRA)
    def _():
        base_e = (_GM * _NW + w) * _CH
        cd2 = pltpu.async_copy(dst_hbm.at[pl.ds(base_e, _CH)],
                               idx_d.at[pl.ds(0, _CH)], sem_i)
        cs2 = pltpu.async_copy(src_hbm.at[pl.ds(base_e, _CH)],
                               idx_s.at[pl.ds(0, _CH)], sem_i)
        cd2.wait()
        cs2.wait()
        isl = pl.ds(0, _CH)
        pltpu.async_copy(t_hbm.at[idx_d.at[isl]], rows_d0, sem_g0).wait()
        pltpu.async_copy(t_hbm.at[idx_s.at[isl]], rows_s0, sem_g0).wait()

        def add_row(j, carry2):
            for cc in range(_H // 16):
                lo = pl.ds(cc * 16, 16)
                hi = pl.ds(_H + cc * 16, 16)
                g_buf0[j, lo] = rows_d0[2 * j, lo] + rows_s0[2 * j, hi]
                g_buf0[j, hi] = rows_d0[2 * j + 1, lo] + rows_s0[2 * j + 1, hi]
            return carry2

        lax.fori_loop(0, _CHP, add_row, 0, unroll=4)
        pltpu.sync_copy(g_buf0,
                        g_hbm.at[pl.ds((_GM * _NW + w) * _CHP, _CHP)])


def _sc_gather(tbl, dst, src):
    return pl.kernel(
        _gather_body,
        out_type=jax.ShapeDtypeStruct((_EP, _HP), _f32),
        mesh=_sc_mesh(),
        scratch_types=[
            pltpu.VMEM((_GIDX,), jnp.int32),
            pltpu.VMEM((_GIDX,), jnp.int32),
            pltpu.VMEM((_CH, _HP), _f32),
            pltpu.VMEM((_CH, _HP), _f32),
            pltpu.VMEM((_CH, _HP), _f32),
            pltpu.VMEM((_CH, _HP), _f32),
            pltpu.VMEM((_CHP, _HP), _f32),
            pltpu.VMEM((_CHP, _HP), _f32),
            pltpu.SemaphoreType.DMA,
            pltpu.SemaphoreType.DMA,
            pltpu.SemaphoreType.DMA,
        ],
    )(tbl, dst, src)


_CE = 256 * _H
_FL = _NP * _H
_NSCH = _E * _H // _CE
_SM = _NSCH // _NW
_SEXTRA = _NSCH - _SM * _NW


def _scatter_body(v_hbm, ei_hbm, out_hbm, agg_sh, ebuf0, ebuf1, vbuf0,
                  vbuf1, sem0, sem1):
    w = _worker_id()
    s = lax.axis_index("s")
    c = lax.axis_index("c")
    sems = (sem0, sem1)
    ebufs = (ebuf0, ebuf1)
    vbufs = (vbuf0, vbuf1)

    def zero_vec(j, carry):
        vbuf0[pl.ds(j * 16, 16)] = jnp.zeros((16,), _f32)
        return carry

    lax.fori_loop(0, _CE // 16, zero_vec, 0, unroll=8)
    eps = _FL // _NS
    nfull, rem = divmod(eps, _CE)
    for r in range(nfull):
        pltpu.sync_copy(vbuf0, agg_sh.at[pl.ds(s * eps + r * _CE, _CE)])
    if rem:
        pltpu.sync_copy(vbuf0.at[pl.ds(0, rem)],
                        agg_sh.at[pl.ds(s * eps + nfull * _CE, rem)])
    plsc.subcore_barrier()

    def scat(b):
        pltpu.make_async_copy(ei_hbm.at[pl.ds(0, _CE)], ebufs[b],
                              sems[b]).wait()
        pltpu.make_async_copy(v_hbm.at[pl.ds(0, _CE)], vbufs[b],
                              sems[b]).wait()
        pltpu.sync_copy(vbufs[b], agg_sh.at[ebufs[b]], add=True)

    def pair(t2, carry):
        for b in range(2):
            cid = w * _SM + t2 * 2 + b

            @pl.when(t2 * 2 + b < _SM)
            def _():
                base = cid * _CE
                pltpu.async_copy(ei_hbm.at[pl.ds(base, _CE)], ebufs[b],
                                 sems[b])
                pltpu.async_copy(v_hbm.at[pl.ds(base, _CE)], vbufs[b],
                                 sems[b])
        for b in range(2):
            @pl.when(t2 * 2 + b < _SM)
            def _(b=b):
                scat(b)
        return carry

    lax.fori_loop(0, (_SM + 1) // 2, pair, 0)

    @pl.when(w < _SEXTRA)
    def _():
        base = (_SM * _NW + w) * _CE
        pltpu.async_copy(ei_hbm.at[pl.ds(base, _CE)], ebuf0, sem0)
        pltpu.async_copy(v_hbm.at[pl.ds(base, _CE)], vbuf0, sem0)
        scat(0)

    plsc.subcore_barrier()
    el_sl = pl.ds(s * eps, eps)
    pltpu.sync_copy(agg_sh.at[el_sl], out_hbm.at[c, el_sl])


def _sc_scatter(v_flat, ei_flat):
    return pl.kernel(
        _scatter_body,
        out_type=jax.ShapeDtypeStruct((_NC, _FL), _f32),
        mesh=_sc_mesh(),
        scratch_types=[
            pltpu.VMEM_SHARED((_FL,), _f32),
            pltpu.VMEM((_CE,), jnp.int32),
            pltpu.VMEM((_CE,), jnp.int32),
            pltpu.VMEM((_CE,), _f32),
            pltpu.VMEM((_CE,), _f32),
            pltpu.SemaphoreType.DMA,
            pltpu.SemaphoreType.DMA,
        ],
    )(v_flat, ei_flat)


def _eidx(dst2):
    def body(d_ref, o_ref):
        d = d_ref[...]
        iota = lax.broadcasted_iota(jnp.int32, (_BEP, _H), 1)
        lo = d[:, 0:1] * _H + iota
        hi = d[:, 1:2] * _H + iota
        o_ref[...] = jnp.concatenate([lo, hi], axis=1)

    return pl.pallas_call(
        body,
        grid=(_EP // _BEP,),
        in_specs=[pl.BlockSpec((_BEP, 2), lambda i: (i, 0))],
        out_specs=_row_spec(_BEP, _HP),
        out_shape=jax.ShapeDtypeStruct((_EP, _HP), jnp.int32),
    )(dst2)


def _dot(a, b):
    return jnp.dot(a, b, preferred_element_type=_f32)


def _bd(wmat):
    k, n = wmat.shape
    out = jnp.zeros((2 * k, 2 * n), _f32)
    out = out.at[:k, :n].set(wmat)
    return out.at[k:, n:].set(wmat)


def _row_spec(bn, d):
    return pl.BlockSpec((bn, d), lambda i: (i, 0))


def _w_spec(k, n):
    return pl.BlockSpec((k, n), lambda i: (0, 0))


def _enc_nodes(xp, w1, b1, w2, b2, wa, ba, wb):
    def body(x_ref, w1_ref, b1_ref, w2_ref, b2_ref, wa_ref, ba_ref, wb_ref,
             h_ref, hd_ref, hs_ref):
        h1 = jnp.maximum(_dot(x_ref[...], w1_ref[...]) + b1_ref[...], 0.0)
        h = jnp.maximum(_dot(h1, w2_ref[...]) + b2_ref[...], 0.0)
        h_ref[...] = h
        hd_ref[...] = _dot(h, wa_ref[...]) + ba_ref[...]
        hs_ref[...] = _dot(h, wb_ref[...])

    return pl.pallas_call(
        body,
        grid=(_NP // _BN,),
        in_specs=[_row_spec(_BN, 128), _w_spec(128, _H), _w_spec(1, _H),
                  _w_spec(_H, _H), _w_spec(1, _H), _w_spec(_H, _H),
                  _w_spec(1, _H), _w_spec(_H, _H)],
        out_specs=[_row_spec(_BN, _H)] * 3,
        out_shape=[jax.ShapeDtypeStruct((_NP, _H), _f32)] * 3,
    )(xp, w1, b1, w2, b2, wa, ba, wb)


def _enc_edges(ea2, w1, b1, w2, b2):
    def body(ea_ref, w1_ref, b1_ref, w2_ref, b2_ref, e_ref):
        h1 = jnp.maximum(_dot(ea_ref[...], w1_ref[...]) + b1_ref[...], 0.0)
        e_ref[...] = jnp.maximum(_dot(h1, w2_ref[...]) + b2_ref[...], 0.0)

    return pl.pallas_call(
        body,
        grid=(_EP // _BEP,),
        in_specs=[_row_spec(_BEP, 32), _w_spec(32, _HP), _w_spec(1, _HP),
                  _w_spec(_HP, _HP), _w_spec(1, _HP)],
        out_specs=_row_spec(_BEP, _HP),
        out_shape=jax.ShapeDtypeStruct((_EP, _HP), _f32),
    )(ea2, w1, b1, w2, b2)


def _msg(g2, e2, w1c, w2, b2):
    def body(g_ref, e_ref, w1c_ref, w2_ref, b2_ref, m_ref):
        t = jnp.maximum(g_ref[...] + _dot(e_ref[...], w1c_ref[...]), 0.0)
        m_ref[...] = jnp.maximum(_dot(t, w2_ref[...]) + b2_ref[...], 0.0)

    return pl.pallas_call(
        body,
        grid=(_EP // _BEP,),
        in_specs=[_row_spec(_BEP, _HP), _row_spec(_BEP, _HP),
                  _w_spec(_HP, _HP), _w_spec(_HP, _HP), _w_spec(1, _HP)],
        out_specs=_row_spec(_BEP, _HP),
        out_shape=jax.ShapeDtypeStruct((_EP, _HP), _f32),
    )(g2, e2, w1c, w2, b2)


def _prep_inv(cntp):
    def body(c_ref, inv_ref):
        inv_ref[...] = 1.0 / jnp.maximum(c_ref[0] + c_ref[1], 1.0)

    return pl.pallas_call(
        body,
        grid=(_NP // _BN,),
        in_specs=[pl.BlockSpec((_NC, _BN, _H), lambda i: (0, i, 0))],
        out_specs=_row_spec(_BN, _H),
        out_shape=jax.ShapeDtypeStruct((_NP, _H), _f32),
    )(cntp)


def _update(h, aggp, inv, u1a, u1b, c1, u2, c2, nxt):
    emit_next = nxt is not None

    def body(h_ref, a_ref, inv_ref, u1a_ref, u1b_ref, c1_ref, u2_ref,
             c2_ref, *rest):
        if emit_next:
            wa_ref, ba_ref, wb_ref, hn_ref, hd_ref, hs_ref = rest
        else:
            (hn_ref,) = rest
        agg = (a_ref[0] + a_ref[1]) * inv_ref[...]
        v = jnp.maximum(_dot(h_ref[...], u1a_ref[...])
                        + _dot(agg, u1b_ref[...]) + c1_ref[...], 0.0)
        u = jnp.maximum(_dot(v, u2_ref[...]) + c2_ref[...], 0.0)
        hn = h_ref[...] + u
        hn_ref[...] = hn
        if emit_next:
            hd_ref[...] = _dot(hn, wa_ref[...]) + ba_ref[...]
            hs_ref[...] = _dot(hn, wb_ref[...])

    in_specs = [_row_spec(_BN, _H),
                pl.BlockSpec((_NC, _BN, _H), lambda i: (0, i, 0)),
                _row_spec(_BN, _H), _w_spec(_H, _H), _w_spec(_H, _H),
                _w_spec(1, _H), _w_spec(_H, _H), _w_spec(1, _H)]
    args = [h, aggp, inv, u1a, u1b, c1, u2, c2]
    if emit_next:
        in_specs += [_w_spec(_H, _H), _w_spec(1, _H), _w_spec(_H, _H)]
        args += list(nxt)
        out_specs = [_row_spec(_BN, _H)] * 3
        out_shape = [jax.ShapeDtypeStruct((_NP, _H), _f32)] * 3
    else:
        out_specs = [_row_spec(_BN, _H)]
        out_shape = [jax.ShapeDtypeStruct((_NP, _H), _f32)]
    out = pl.pallas_call(
        body,
        grid=(_NP // _BN,),
        in_specs=in_specs,
        out_specs=out_specs,
        out_shape=out_shape,
    )(*args)
    return out if emit_next else (out[0], None, None)


def _head(h, c1m, b1r, c2m, b2r):
    def body(h_ref, c1_ref, b1_ref, c2_ref, b2_ref, o_ref):
        z1 = jnp.maximum(_dot(h_ref[...], c1_ref[...]) + b1_ref[...], 0.0)
        o_ref[...] = _dot(z1, c2_ref[...]) + b2_ref[...]

    return pl.pallas_call(
        body,
        grid=(_NP // _BN,),
        in_specs=[_row_spec(_BN, _H), _w_spec(_H, 104), _w_spec(1, 104),
                  _w_spec(104, 4), _w_spec(1, 4)],
        out_specs=_row_spec(_BN, 4),
        out_shape=jax.ShapeDtypeStruct((_NP, 4), _f32),
    )(h, c1m, b1r, c2m, b2r)


def kernel(x, edge_index, edge_attr, enc_n_W1, enc_n_b1, enc_n_W2, enc_n_b2,
           enc_e_W1, enc_e_b1, enc_e_W2, enc_e_b2, msg_W1, msg_b1, msg_W2,
           msg_b2, upd_W1, upd_b1, upd_W2, upd_b2, conv1_w, conv1_b,
           conv2_w, conv2_b):
    L = msg_W1.shape[0]
    src = edge_index[0].astype(jnp.int32)
    dst = edge_index[1].astype(jnp.int32)
    xp = jnp.pad(x, ((0, _NP - _N), (0, 0)))
    ea2 = edge_attr.reshape(_EP, 32)

    W1a = msg_W1[:, :_H, :]
    W1b = msg_W1[:, _H:2 * _H, :]
    W1c = msg_W1[:, 2 * _H:, :]
    U1a = upd_W1[:, :_H, :]
    U1b = upd_W1[:, _H:, :]
    r1 = lambda v: v.reshape(1, -1)
    r2 = lambda v: jnp.concatenate([v, v]).reshape(1, -1)

    w1t = conv1_w[:, 0, :].T
    M1 = jnp.zeros((_H, 8, 13), _f32)
    for j in range(13):
        M1 = M1.at[4 * j:4 * j + 15, :, j].set(w1t)
    C1 = M1.reshape(_H, 104)
    B1 = jnp.repeat(conv1_b, 13).reshape(1, 104)
    M2 = jnp.zeros((8, 13, 4), _f32)
    for t in range(4):
        M2 = M2.at[:, t:t + 10, t].set(conv2_w[0])
    C2 = M2.reshape(104, 4)
    B2 = jnp.broadcast_to(conv2_b.reshape(1, 1), (1, 4))

    e2 = _enc_edges(ea2, _bd(enc_e_W1), r2(enc_e_b1), _bd(enc_e_W2),
                    r2(enc_e_b2))
    h, hd, hs = _enc_nodes(xp, enc_n_W1, r1(enc_n_b1), enc_n_W2,
                           r1(enc_n_b2), W1a[0], r1(msg_b1[0]), W1b[0])

    ei_flat = _eidx(dst.reshape(_EP, 2)).reshape(_EP * _HP)
    ones_flat = jnp.ones((_EP * _HP,), _f32)
    cntp = _sc_scatter(ones_flat, ei_flat).reshape(_NC, _NP, _H)
    inv = _prep_inv(cntp)

    for l in range(L):
        g2 = _sc_gather(jnp.concatenate([hd, hs], axis=1), dst, src)
        m2 = _msg(g2, e2, _bd(W1c[l]), _bd(msg_W2[l]), r2(msg_b2[l]))
        aggp = _sc_scatter(m2.reshape(_EP * _HP), ei_flat).reshape(
            _NC, _NP, _H)
        nxt = (W1a[l + 1], r1(msg_b1[l + 1]), W1b[l + 1]) if l + 1 < L else None
        h, hd, hs = _update(h, aggp, inv, U1a[l], U1b[l], r1(upd_b1[l]),
                            upd_W2[l], r1(upd_b2[l]), nxt)

    out = _head(h, C1, B1, C2, B2)
    return out[:_N]


# speedup vs baseline: 1.0191x; 1.0191x over previous
import jax
import jax.numpy as jnp
from jax import lax
from jax.experimental import pallas as pl
from jax.experimental.pallas import tpu as pltpu
from jax.experimental.pallas import tpu_sc as plsc

_N = 10000
_NP = 10240
_E = 160000
_EP = _E // 2
_H = 64
_HP = 128
_NC = 2
_NS = 16
_NW = _NC * _NS
_CH = 128
_CHP = _CH // 2
_NCHUNKS = _E // _CH
_BN = 2048
_BEP = 4000

_f32 = jnp.float32


def _sc_mesh():
    return plsc.VectorSubcoreMesh(
        core_axis_name="c", subcore_axis_name="s",
        num_cores=_NC, num_subcores=_NS)


_GM = _NCHUNKS // _NW
_GEXTRA = _NCHUNKS - _GM * _NW
_GIDX = _GM * _CH


def _worker_id():
    return lax.axis_index("s") * _NC + lax.axis_index("c")


def _gather_body(t_hbm, dst_hbm, src_hbm, g_hbm,
                 idx_d, idx_s, rows_d0, rows_d1, rows_s0, rows_s1,
                 g_buf0, g_buf1, sem_i, sem_g0, sem_g1):
    w = _worker_id()
    start_e = w * _GIDX
    cd = pltpu.async_copy(dst_hbm.at[pl.ds(start_e, _GIDX)], idx_d, sem_i)
    cs = pltpu.async_copy(src_hbm.at[pl.ds(start_e, _GIDX)], idx_s, sem_i)
    cd.wait()
    cs.wait()
    sems = (sem_g0, sem_g1)
    rows_d = (rows_d0, rows_d1)
    rows_s = (rows_s0, rows_s1)
    g_bufs = (g_buf0, g_buf1)

    def add_chunk(b, c):
        rd, rs, gb = rows_d[b], rows_s[b], g_bufs[b]

        def add_row(j, carry2):
            for cc in range(_H // 16):
                lo = pl.ds(cc * 16, 16)
                hi = pl.ds(_H + cc * 16, 16)
                gb[j, lo] = rd[2 * j, lo] + rs[2 * j, hi]
                gb[j, hi] = rd[2 * j + 1, lo] + rs[2 * j + 1, hi]
            return carry2

        lax.fori_loop(0, _CHP, add_row, 0, unroll=4)
        base_p = (w * _GM + c) * _CHP
        pltpu.sync_copy(gb, g_hbm.at[pl.ds(base_p, _CHP)])

    def pair(t2, carry):
        for b in range(2):
            c = t2 * 2 + b

            @pl.when(c < _GM)
            def _():
                isl = pl.ds(c * _CH, _CH)
                pltpu.async_copy(t_hbm.at[idx_d.at[isl]], rows_d[b], sems[b])
                pltpu.async_copy(t_hbm.at[idx_s.at[isl]], rows_s[b], sems[b])
        for b in range(2):
            c = t2 * 2 + b

            @pl.when(c < _GM)
            def _():
                pltpu.make_async_copy(t_hbm.at[idx_d.at[pl.ds(0, _CH)]],
                                      rows_d[b], sems[b]).wait()
                pltpu.make_async_copy(t_hbm.at[idx_s.at[pl.ds(0, _CH)]],
                                      rows_s[b], sems[b]).wait()
                add_chunk(b, c)
        return carry

    lax.fori_loop(0, (_GM + 1) // 2, pair, 0)

    @pl.when(w < _GEXTRA)
    def _():
        base_e = (_GM * _NW + w) * _CH
        cd2 = pltpu.async_copy(dst_hbm.at[pl.ds(base_e, _CH)],
                               idx_d.at[pl.ds(0, _CH)], sem_i)
        cs2 = pltpu.async_copy(src_hbm.at[pl.ds(base_e, _CH)],
                               idx_s.at[pl.ds(0, _CH)], sem_i)
        cd2.wait()
        cs2.wait()
        isl = pl.ds(0, _CH)
        pltpu.async_copy(t_hbm.at[idx_d.at[isl]], rows_d0, sem_g0).wait()
        pltpu.async_copy(t_hbm.at[idx_s.at[isl]], rows_s0, sem_g0).wait()

        def add_row(j, carry2):
            for cc in range(_H // 16):
                lo = pl.ds(cc * 16, 16)
                hi = pl.ds(_H + cc * 16, 16)
                g_buf0[j, lo] = rows_d0[2 * j, lo] + rows_s0[2 * j, hi]
                g_buf0[j, hi] = rows_d0[2 * j + 1, lo] + rows_s0[2 * j + 1, hi]
            return carry2

        lax.fori_loop(0, _CHP, add_row, 0, unroll=4)
        pltpu.sync_copy(g_buf0,
                        g_hbm.at[pl.ds((_GM * _NW + w) * _CHP, _CHP)])


def _sc_gather(tbl, dst, src):
    return pl.kernel(
        _gather_body,
        out_type=jax.ShapeDtypeStruct((_EP, _HP), _f32),
        mesh=_sc_mesh(),
        scratch_types=[
            pltpu.VMEM((_GIDX,), jnp.int32),
            pltpu.VMEM((_GIDX,), jnp.int32),
            pltpu.VMEM((_CH, _HP), _f32),
            pltpu.VMEM((_CH, _HP), _f32),
            pltpu.VMEM((_CH, _HP), _f32),
            pltpu.VMEM((_CH, _HP), _f32),
            pltpu.VMEM((_CHP, _HP), _f32),
            pltpu.VMEM((_CHP, _HP), _f32),
            pltpu.SemaphoreType.DMA,
            pltpu.SemaphoreType.DMA,
            pltpu.SemaphoreType.DMA,
        ],
    )(tbl, dst, src)


_CE = 256 * _H
_FL = _NP * _H
_NSCH = _E * _H // _CE
_SM = _NSCH // _NW
_SEXTRA = _NSCH - _SM * _NW


def _scatter_body(v_hbm, ei_hbm, out_hbm, agg_sh, ebuf0, ebuf1, vbuf0,
                  vbuf1, sem0, sem1):
    w = _worker_id()
    s = lax.axis_index("s")
    c = lax.axis_index("c")
    sems = (sem0, sem1)
    ebufs = (ebuf0, ebuf1)
    vbufs = (vbuf0, vbuf1)

    def zero_vec(j, carry):
        vbuf0[pl.ds(j * 16, 16)] = jnp.zeros((16,), _f32)
        return carry

    lax.fori_loop(0, _CE // 16, zero_vec, 0, unroll=8)
    eps = _FL // _NS
    nfull, rem = divmod(eps, _CE)
    for r in range(nfull):
        pltpu.sync_copy(vbuf0, agg_sh.at[pl.ds(s * eps + r * _CE, _CE)])
    if rem:
        pltpu.sync_copy(vbuf0.at[pl.ds(0, rem)],
                        agg_sh.at[pl.ds(s * eps + nfull * _CE, rem)])
    plsc.subcore_barrier()

    def scat(b):
        pltpu.make_async_copy(ei_hbm.at[pl.ds(0, _CE)], ebufs[b],
                              sems[b]).wait()
        pltpu.make_async_copy(v_hbm.at[pl.ds(0, _CE)], vbufs[b],
                              sems[b]).wait()
        pltpu.sync_copy(vbufs[b], agg_sh.at[ebufs[b]], add=True)

    def pair(t2, carry):
        for b in range(2):
            cid = w * _SM + t2 * 2 + b

            @pl.when(t2 * 2 + b < _SM)
            def _():
                base = cid * _CE
                pltpu.async_copy(ei_hbm.at[pl.ds(base, _CE)], ebufs[b],
                                 sems[b])
                pltpu.async_copy(v_hbm.at[pl.ds(base, _CE)], vbufs[b],
                                 sems[b])
        for b in range(2):
            @pl.when(t2 * 2 + b < _SM)
            def _(b=b):
                scat(b)
        return carry

    lax.fori_loop(0, (_SM + 1) // 2, pair, 0)

    @pl.when(w < _SEXTRA)
    def _():
        base = (_SM * _NW + w) * _CE
        pltpu.async_copy(ei_hbm.at[pl.ds(base, _CE)], ebuf0, sem0)
        pltpu.async_copy(v_hbm.at[pl.ds(base, _CE)], vbuf0, sem0)
        scat(0)

    plsc.subcore_barrier()
    el_sl = pl.ds(s * eps, eps)
    pltpu.sync_copy(agg_sh.at[el_sl], out_hbm.at[c, el_sl])


def _sc_scatter(v_flat, ei_flat):
    return pl.kernel(
        _scatter_body,
        out_type=jax.ShapeDtypeStruct((_NC, _FL), _f32),
        mesh=_sc_mesh(),
        scratch_types=[
            pltpu.VMEM_SHARED((_FL,), _f32),
            pltpu.VMEM((_CE,), jnp.int32),
            pltpu.VMEM((_CE,), jnp.int32),
            pltpu.VMEM((_CE,), _f32),
            pltpu.VMEM((_CE,), _f32),
            pltpu.SemaphoreType.DMA,
            pltpu.SemaphoreType.DMA,
        ],
    )(v_flat, ei_flat)


def _eidx(dst2):
    def body(d_ref, o_ref):
        d = d_ref[...]
        iota = lax.broadcasted_iota(jnp.int32, (_BEP, _H), 1)
        lo = d[:, 0:1] * _H + iota
        hi = d[:, 1:2] * _H + iota
        o_ref[...] = jnp.concatenate([lo, hi], axis=1)

    return pl.pallas_call(
        body,
        grid=(_EP // _BEP,),
        in_specs=[pl.BlockSpec((_BEP, 2), lambda i: (i, 0))],
        out_specs=_row_spec(_BEP, _HP),
        out_shape=jax.ShapeDtypeStruct((_EP, _HP), jnp.int32),
    )(dst2)


def _dot(a, b):
    return jnp.dot(a, b, preferred_element_type=_f32)


def _bd(wmat):
    k, n = wmat.shape
    out = jnp.zeros((2 * k, 2 * n), _f32)
    out = out.at[:k, :n].set(wmat)
    return out.at[k:, n:].set(wmat)


def _row_spec(bn, d):
    return pl.BlockSpec((bn, d), lambda i: (i, 0))


def _w_spec(k, n):
    return pl.BlockSpec((k, n), lambda i: (0, 0))


def _enc_nodes(xp, w1, b1, w2, b2, wa, ba, wb):
    def body(x_ref, w1_ref, b1_ref, w2_ref, b2_ref, wa_ref, ba_ref, wb_ref,
             h_ref, t_ref):
        h1 = jnp.maximum(_dot(x_ref[...], w1_ref[...]) + b1_ref[...], 0.0)
        h = jnp.maximum(_dot(h1, w2_ref[...]) + b2_ref[...], 0.0)
        h_ref[...] = h
        hd = _dot(h, wa_ref[...]) + ba_ref[...]
        hs = _dot(h, wb_ref[...])
        t_ref[...] = jnp.concatenate([hd, hs], axis=1)

    return pl.pallas_call(
        body,
        grid=(_NP // _BN,),
        in_specs=[_row_spec(_BN, 128), _w_spec(128, _H), _w_spec(1, _H),
                  _w_spec(_H, _H), _w_spec(1, _H), _w_spec(_H, _H),
                  _w_spec(1, _H), _w_spec(_H, _H)],
        out_specs=[_row_spec(_BN, _H), _row_spec(_BN, _HP)],
        out_shape=[jax.ShapeDtypeStruct((_NP, _H), _f32),
                   jax.ShapeDtypeStruct((_NP, _HP), _f32)],
    )(xp, w1, b1, w2, b2, wa, ba, wb)


def _enc_edges(ea2, w1, b1, w2, b2):
    def body(ea_ref, w1_ref, b1_ref, w2_ref, b2_ref, e_ref):
        h1 = jnp.maximum(_dot(ea_ref[...], w1_ref[...]) + b1_ref[...], 0.0)
        e_ref[...] = jnp.maximum(_dot(h1, w2_ref[...]) + b2_ref[...], 0.0)

    return pl.pallas_call(
        body,
        grid=(_EP // _BEP,),
        in_specs=[_row_spec(_BEP, 32), _w_spec(32, _HP), _w_spec(1, _HP),
                  _w_spec(_HP, _HP), _w_spec(1, _HP)],
        out_specs=_row_spec(_BEP, _HP),
        out_shape=jax.ShapeDtypeStruct((_EP, _HP), _f32),
    )(ea2, w1, b1, w2, b2)


def _msg(g2, e2, w1c, w2, b2):
    def body(g_ref, e_ref, w1c_ref, w2_ref, b2_ref, m_ref):
        t = jnp.maximum(g_ref[...] + _dot(e_ref[...], w1c_ref[...]), 0.0)
        m_ref[...] = jnp.maximum(_dot(t, w2_ref[...]) + b2_ref[...], 0.0)

    return pl.pallas_call(
        body,
        grid=(_EP // _BEP,),
        in_specs=[_row_spec(_BEP, _HP), _row_spec(_BEP, _HP),
                  _w_spec(_HP, _HP), _w_spec(_HP, _HP), _w_spec(1, _HP)],
        out_specs=_row_spec(_BEP, _HP),
        out_shape=jax.ShapeDtypeStruct((_EP, _HP), _f32),
    )(g2, e2, w1c, w2, b2)


def _prep_inv(cntp):
    def body(c_ref, inv_ref):
        inv_ref[...] = 1.0 / jnp.maximum(c_ref[0] + c_ref[1], 1.0)

    return pl.pallas_call(
        body,
        grid=(_NP // _BN,),
        in_specs=[pl.BlockSpec((_NC, _BN, _H), lambda i: (0, i, 0))],
        out_specs=_row_spec(_BN, _H),
        out_shape=jax.ShapeDtypeStruct((_NP, _H), _f32),
    )(cntp)


def _update(h, aggp, inv, u1a, u1b, c1, u2, c2, nxt):
    emit_next = nxt is not None

    def body(h_ref, a_ref, inv_ref, u1a_ref, u1b_ref, c1_ref, u2_ref,
             c2_ref, *rest):
        if emit_next:
            wa_ref, ba_ref, wb_ref, hn_ref, t_ref = rest
        else:
            (hn_ref,) = rest
        agg = (a_ref[0] + a_ref[1]) * inv_ref[...]
        v = jnp.maximum(_dot(h_ref[...], u1a_ref[...])
                        + _dot(agg, u1b_ref[...]) + c1_ref[...], 0.0)
        u = jnp.maximum(_dot(v, u2_ref[...]) + c2_ref[...], 0.0)
        hn = h_ref[...] + u
        hn_ref[...] = hn
        if emit_next:
            hd = _dot(hn, wa_ref[...]) + ba_ref[...]
            hs = _dot(hn, wb_ref[...])
            t_ref[...] = jnp.concatenate([hd, hs], axis=1)

    in_specs = [_row_spec(_BN, _H),
                pl.BlockSpec((_NC, _BN, _H), lambda i: (0, i, 0)),
                _row_spec(_BN, _H), _w_spec(_H, _H), _w_spec(_H, _H),
                _w_spec(1, _H), _w_spec(_H, _H), _w_spec(1, _H)]
    args = [h, aggp, inv, u1a, u1b, c1, u2, c2]
    if emit_next:
        in_specs += [_w_spec(_H, _H), _w_spec(1, _H), _w_spec(_H, _H)]
        args += list(nxt)
        out_specs = [_row_spec(_BN, _H), _row_spec(_BN, _HP)]
        out_shape = [jax.ShapeDtypeStruct((_NP, _H), _f32),
                     jax.ShapeDtypeStruct((_NP, _HP), _f32)]
    else:
        out_specs = [_row_spec(_BN, _H)]
        out_shape = [jax.ShapeDtypeStruct((_NP, _H), _f32)]
    out = pl.pallas_call(
        body,
        grid=(_NP // _BN,),
        in_specs=in_specs,
        out_specs=out_specs,
        out_shape=out_shape,
    )(*args)
    return out if emit_next else (out[0], None)


def _head(h, c1m, b1r, c2m, b2r):
    def body(h_ref, c1_ref, b1_ref, c2_ref, b2_ref, o_ref):
        z1 = jnp.maximum(_dot(h_ref[...], c1_ref[...]) + b1_ref[...], 0.0)
        o_ref[...] = _dot(z1, c2_ref[...]) + b2_ref[...]

    return pl.pallas_call(
        body,
        grid=(_NP // _BN,),
        in_specs=[_row_spec(_BN, _H), _w_spec(_H, 104), _w_spec(1, 104),
                  _w_spec(104, 4), _w_spec(1, 4)],
        out_specs=_row_spec(_BN, 4),
        out_shape=jax.ShapeDtypeStruct((_NP, 4), _f32),
    )(h, c1m, b1r, c2m, b2r)


def kernel(x, edge_index, edge_attr, enc_n_W1, enc_n_b1, enc_n_W2, enc_n_b2,
           enc_e_W1, enc_e_b1, enc_e_W2, enc_e_b2, msg_W1, msg_b1, msg_W2,
           msg_b2, upd_W1, upd_b1, upd_W2, upd_b2, conv1_w, conv1_b,
           conv2_w, conv2_b):
    L = msg_W1.shape[0]
    src = edge_index[0].astype(jnp.int32)
    dst = edge_index[1].astype(jnp.int32)
    xp = jnp.pad(x, ((0, _NP - _N), (0, 0)))
    ea2 = edge_attr.reshape(_EP, 32)

    W1a = msg_W1[:, :_H, :]
    W1b = msg_W1[:, _H:2 * _H, :]
    W1c = msg_W1[:, 2 * _H:, :]
    U1a = upd_W1[:, :_H, :]
    U1b = upd_W1[:, _H:, :]
    r1 = lambda v: v.reshape(1, -1)
    r2 = lambda v: jnp.concatenate([v, v]).reshape(1, -1)

    w1t = conv1_w[:, 0, :].T
    M1 = jnp.zeros((_H, 8, 13), _f32)
    for j in range(13):
        M1 = M1.at[4 * j:4 * j + 15, :, j].set(w1t)
    C1 = M1.reshape(_H, 104)
    B1 = jnp.repeat(conv1_b, 13).reshape(1, 104)
    M2 = jnp.zeros((8, 13, 4), _f32)
    for t in range(4):
        M2 = M2.at[:, t:t + 10, t].set(conv2_w[0])
    C2 = M2.reshape(104, 4)
    B2 = jnp.broadcast_to(conv2_b.reshape(1, 1), (1, 4))

    e2 = _enc_edges(ea2, _bd(enc_e_W1), r2(enc_e_b1), _bd(enc_e_W2),
                    r2(enc_e_b2))
    h, tbl = _enc_nodes(xp, enc_n_W1, r1(enc_n_b1), enc_n_W2,
                        r1(enc_n_b2), W1a[0], r1(msg_b1[0]), W1b[0])

    ei_flat = _eidx(dst.reshape(_EP, 2)).reshape(_EP * _HP)
    ones_flat = jnp.ones((_EP * _HP,), _f32)
    cntp = _sc_scatter(ones_flat, ei_flat).reshape(_NC, _NP, _H)
    inv = _prep_inv(cntp)

    for l in range(L):
        g2 = _sc_gather(tbl, dst, src)
        m2 = _msg(g2, e2, _bd(W1c[l]), _bd(msg_W2[l]), r2(msg_b2[l]))
        aggp = _sc_scatter(m2.reshape(_EP * _HP), ei_flat).reshape(
            _NC, _NP, _H)
        nxt = (W1a[l + 1], r1(msg_b1[l + 1]), W1b[l + 1]) if l + 1 < L else None
        h, tbl = _update(h, aggp, inv, U1a[l], U1b[l], r1(upd_b1[l]),
                         upd_W2[l], r1(upd_b2[l]), nxt)

    out = _head(h, C1, B1, C2, B2)
    return out[:_N]
